# baseline (speedup 1.0000x reference)
import functools

import jax
import jax.numpy as jnp
from jax import lax
from jax.experimental import pallas as pl
from jax.experimental.pallas import tpu as pltpu
from jax.experimental.pallas import tpu_sc as plsc

N = 10000
F = 128
H = 64
O = 64
G = 4
E = 320000

NC, NS = 2, 16
NW = NC * NS
CH = 128
EPT = 10240
NCHUNK = EPT // CH
NBUF = 4
E_PAD = NW * EPT
N_ACC = 10240
RPT = N_ACC // NS

ROW_BLK = 2000


def _lin_in_body(x_ref, w_ref, o_ref):
    o_ref[...] = jnp.dot(x_ref[...], w_ref[...],
                         preferred_element_type=jnp.float32)


def _lin_in(x, W_in):
    return pl.pallas_call(
        _lin_in_body,
        grid=(N // ROW_BLK,),
        in_specs=[
            pl.BlockSpec((ROW_BLK, F), lambda i: (i, 0)),
            pl.BlockSpec((F, H), lambda i: (0, 0)),
        ],
        out_specs=pl.BlockSpec((ROW_BLK, H), lambda i: (i, 0)),
        out_shape=jax.ShapeDtypeStruct((N, H), jnp.float32),
    )(x, W_in)


def _make_spmm():
    mesh = plsc.VectorSubcoreMesh(core_axis_name="c", subcore_axis_name="s",
                                  num_cores=NC, num_subcores=NS)

    @functools.partial(
        pl.kernel,
        out_type=jax.ShapeDtypeStruct((NC, N_ACC, H), jnp.float32),
        mesh=mesh,
        scratch_types=[
            pltpu.VMEM((NCHUNK + NBUF, CH), jnp.int32),
            pltpu.VMEM((NCHUNK, CH), jnp.int32),
            pltpu.VMEM((NBUF, CH, H), jnp.float32),
            pltpu.VMEM_SHARED((N_ACC, H), jnp.float32),
            pltpu.SemaphoreType.DMA,
            pltpu.SemaphoreType.DMA,
            pltpu.SemaphoreType.DMA,
            pltpu.SemaphoreType.DMA,
        ],
        compiler_params=pltpu.CompilerParams(use_tc_tiling_on_sc=False),
    )
    def spmm(h_hbm, src_hbm, dst_hbm, zeros_hbm, out_hbm,
             src_v, dst_v, rows_v, acc, s0, s1, s2, s3):
        sems = (s0, s1, s2, s3)
        cid = lax.axis_index("c")
        sid = lax.axis_index("s")
        wid = sid * NC + cid
        pltpu.sync_copy(zeros_hbm, acc.at[pl.ds(sid * RPT, RPT)])
        pltpu.sync_copy(src_hbm.at[wid], src_v)
        pltpu.sync_copy(dst_hbm.at[wid], dst_v)
        plsc.subcore_barrier()

        for b in range(NBUF):
            pltpu.async_copy(h_hbm.at[src_v.at[b]], rows_v.at[b], sems[b])

        def body(i, carry):
            c0 = i * NBUF
            for b in range(NBUF):
                c = c0 + b
                pltpu.make_async_copy(
                    h_hbm.at[src_v.at[c]], rows_v.at[b], sems[b]).wait()
                pltpu.sync_copy(rows_v.at[b], acc.at[dst_v.at[c]], add=True)
                pltpu.async_copy(
                    h_hbm.at[src_v.at[c + NBUF]], rows_v.at[b], sems[b])
            return carry

        lax.fori_loop(0, NCHUNK // NBUF, body, 0)
        for b in range(NBUF):
            pltpu.make_async_copy(
                h_hbm.at[src_v.at[NCHUNK + b]], rows_v.at[b], sems[b]).wait()
        plsc.subcore_barrier()
        pltpu.sync_copy(acc.at[pl.ds(sid * RPT, RPT)],
                        out_hbm.at[cid, pl.ds(sid * RPT, RPT)])

    return spmm


_spmm_cache = []


def _spmm(*args):
    if not _spmm_cache:
        _spmm_cache.append(_make_spmm())
    return _spmm_cache[0](*args)


def _kan_body(p_ref, wc_ref, ws_ref, wo_ref, o_ref):
    agg = p_ref[0] + p_ref[1]
    y = jnp.zeros((ROW_BLK, H), jnp.float32)
    for g in range(G):
        kg = jnp.float32(g + 1)
        y = y + jnp.dot(jnp.cos(kg * agg), wc_ref[g],
                        preferred_element_type=jnp.float32)
        y = y + jnp.dot(jnp.sin(kg * agg), ws_ref[g],
                        preferred_element_type=jnp.float32)
    o = jnp.dot(y, wo_ref[...], preferred_element_type=jnp.float32)
    m = jnp.max(o, axis=-1, keepdims=True)
    lse = jnp.log(jnp.sum(jnp.exp(o - m), axis=-1, keepdims=True))
    o_ref[...] = (o - m) - lse


def _kan_out(partials, Wc, Ws, W_out):
    return pl.pallas_call(
        _kan_body,
        grid=(N // ROW_BLK,),
        in_specs=[
            pl.BlockSpec((NC, ROW_BLK, H), lambda i: (0, i, 0)),
            pl.BlockSpec((G, H, H), lambda i: (0, 0, 0)),
            pl.BlockSpec((G, H, H), lambda i: (0, 0, 0)),
            pl.BlockSpec((H, O), lambda i: (0, 0)),
        ],
        out_specs=pl.BlockSpec((ROW_BLK, O), lambda i: (i, 0)),
        out_shape=jax.ShapeDtypeStruct((N, O), jnp.float32),
    )(partials, Wc, Ws, W_out)


def kernel(x, adj, W_in, kan_coeffs, W_out):
    h = _lin_in(x, W_in)

    n_pad = E_PAD - E
    src = jnp.concatenate(
        [adj[0], jnp.zeros((n_pad,), jnp.int32)]).reshape(NW, NCHUNK, CH)
    src = jnp.concatenate(
        [src, jnp.zeros((NW, NBUF, CH), jnp.int32)], axis=1)
    dst = jnp.concatenate(
        [adj[1], jnp.full((n_pad,), N, jnp.int32)]).reshape(NW, NCHUNK, CH)
    zeros = jnp.zeros((RPT, H), jnp.float32)

    partials = _spmm(h, src, dst, zeros)

    Wc = jnp.transpose(kan_coeffs[0], (2, 1, 0))
    Ws = jnp.transpose(kan_coeffs[1], (2, 1, 0))
    return _kan_out(partials, Wc, Ws, W_out)


# speedup vs baseline: 1.8139x; 1.8139x over previous
import functools

import jax
import jax.numpy as jnp
from jax import lax
from jax.experimental import pallas as pl
from jax.experimental.pallas import tpu as pltpu
from jax.experimental.pallas import tpu_sc as plsc

N = 10000
F = 128
H = 64
O = 64
G = 4
E = 320000

NC, NS = 2, 16
NW = NC * NS
CH = 128
EPT = 10240
NCHUNK = EPT // CH
NBUF = 4
E_PAD = NW * EPT
N_ACC = 10240
RPT = N_ACC // NS

ROW_BLK = 2000


def _lin_in_body(x_ref, w_ref, o_ref):
    o_ref[...] = jnp.dot(x_ref[...], w_ref[...],
                         preferred_element_type=jnp.float32)


def _lin_in(x, W_in):
    return pl.pallas_call(
        _lin_in_body,
        grid=(N // ROW_BLK,),
        in_specs=[
            pl.BlockSpec((ROW_BLK, F), lambda i: (i, 0)),
            pl.BlockSpec((F, H), lambda i: (0, 0)),
        ],
        out_specs=pl.BlockSpec((ROW_BLK, H), lambda i: (i, 0)),
        out_shape=jax.ShapeDtypeStruct((N, H), jnp.float32),
    )(x, W_in)


def _make_spmm():
    mesh = plsc.VectorSubcoreMesh(core_axis_name="c", subcore_axis_name="s",
                                  num_cores=NC, num_subcores=NS)

    @functools.partial(
        pl.kernel,
        out_type=jax.ShapeDtypeStruct((NC, N_ACC, H), jnp.float32),
        mesh=mesh,
        scratch_types=[
            pltpu.VMEM((NCHUNK, CH), jnp.int32),
            pltpu.VMEM((NCHUNK, CH), jnp.int32),
            pltpu.VMEM((NBUF, CH, H), jnp.float32),
            pltpu.VMEM_SHARED((N_ACC, H), jnp.float32),
            pltpu.SemaphoreType.DMA,
            pltpu.SemaphoreType.DMA,
            pltpu.SemaphoreType.DMA,
            pltpu.SemaphoreType.DMA,
        ],
        compiler_params=pltpu.CompilerParams(use_tc_tiling_on_sc=False),
    )
    def spmm(h_hbm, src_hbm, dst_hbm, zeros_hbm, out_hbm,
             src_v, dst_v, rows_v, acc, s0, s1, s2, s3):
        sems = (s0, s1, s2, s3)
        cid = lax.axis_index("c")
        sid = lax.axis_index("s")
        wid = sid * NC + cid
        pltpu.sync_copy(zeros_hbm, acc.at[pl.ds(sid * RPT, RPT)])
        pltpu.sync_copy(src_hbm.at[wid], src_v)
        pltpu.sync_copy(dst_hbm.at[wid], dst_v)
        plsc.subcore_barrier()

        for b in range(NBUF):
            pltpu.async_copy(h_hbm.at[src_v.at[b]], rows_v.at[b], sems[b])

        def body(i, carry):
            c0 = i * NBUF
            for b in range(NBUF):
                c = c0 + b
                pltpu.make_async_copy(
                    h_hbm.at[src_v.at[c]], rows_v.at[b], sems[b]).wait()
                pltpu.sync_copy(rows_v.at[b], acc.at[dst_v.at[c]], add=True)
                pltpu.async_copy(
                    h_hbm.at[src_v.at[c + NBUF]], rows_v.at[b], sems[b])
            return carry

        lax.fori_loop(0, NCHUNK // NBUF - 1, body, 0)

        for b in range(NBUF):
            c = NCHUNK - NBUF + b
            pltpu.make_async_copy(
                h_hbm.at[src_v.at[c]], rows_v.at[b], sems[b]).wait()
            pltpu.sync_copy(rows_v.at[b], acc.at[dst_v.at[c]], add=True)
        plsc.subcore_barrier()
        pltpu.sync_copy(acc.at[pl.ds(sid * RPT, RPT)],
                        out_hbm.at[cid, pl.ds(sid * RPT, RPT)])

    return spmm


_spmm_cache = []


def _spmm(*args):
    if not _spmm_cache:
        _spmm_cache.append(_make_spmm())
    return _spmm_cache[0](*args)


def _kan_body(p_ref, wc_ref, ws_ref, wo_ref, o_ref):
    agg = p_ref[0] + p_ref[1]
    y = jnp.zeros((ROW_BLK, H), jnp.float32)
    for g in range(G):
        kg = jnp.float32(g + 1)
        y = y + jnp.dot(jnp.cos(kg * agg), wc_ref[g],
                        preferred_element_type=jnp.float32)
        y = y + jnp.dot(jnp.sin(kg * agg), ws_ref[g],
                        preferred_element_type=jnp.float32)
    o = jnp.dot(y, wo_ref[...], preferred_element_type=jnp.float32)
    m = jnp.max(o, axis=-1, keepdims=True)
    lse = jnp.log(jnp.sum(jnp.exp(o - m), axis=-1, keepdims=True))
    o_ref[...] = (o - m) - lse


def _kan_out(partials, Wc, Ws, W_out):
    return pl.pallas_call(
        _kan_body,
        grid=(N // ROW_BLK,),
        in_specs=[
            pl.BlockSpec((NC, ROW_BLK, H), lambda i: (0, i, 0)),
            pl.BlockSpec((G, H, H), lambda i: (0, 0, 0)),
            pl.BlockSpec((G, H, H), lambda i: (0, 0, 0)),
            pl.BlockSpec((H, O), lambda i: (0, 0)),
        ],
        out_specs=pl.BlockSpec((ROW_BLK, O), lambda i: (i, 0)),
        out_shape=jax.ShapeDtypeStruct((N, O), jnp.float32),
    )(partials, Wc, Ws, W_out)


def kernel(x, adj, W_in, kan_coeffs, W_out):
    h = _lin_in(x, W_in)

    n_pad = E_PAD - E
    src = jnp.concatenate(
        [adj[0], jnp.zeros((n_pad,), jnp.int32)]).reshape(NW, NCHUNK, CH)
    dst = jnp.concatenate(
        [adj[1], jnp.full((n_pad,), N, jnp.int32)]).reshape(NW, NCHUNK, CH)
    zeros = jnp.zeros((RPT, H), jnp.float32)

    partials = _spmm(h, src, dst, zeros)

    Wc = jnp.transpose(kan_coeffs[0], (2, 1, 0))
    Ws = jnp.transpose(kan_coeffs[1], (2, 1, 0))
    return _kan_out(partials, Wc, Ws, W_out)


# speedup vs baseline: 1.8810x; 1.0370x over previous
import functools

import jax
import jax.numpy as jnp
from jax import lax
from jax.experimental import pallas as pl
from jax.experimental.pallas import tpu as pltpu
from jax.experimental.pallas import tpu_sc as plsc

N = 10000
F = 128
H = 64
O = 64
G = 4
E = 320000

NC, NS = 2, 16
NW = NC * NS
CH = 128
EPT = 10240
NCHUNK = EPT // CH
NBUF = 8
E_PAD = NW * EPT
N_ACC = 10240
RPT = N_ACC // NS

ROW_BLK = 2000


def _lin_in_body(x_ref, w_ref, o_ref):
    o_ref[...] = jnp.dot(x_ref[...], w_ref[...],
                         preferred_element_type=jnp.float32)


def _lin_in(x, W_in):
    return pl.pallas_call(
        _lin_in_body,
        grid=(N // ROW_BLK,),
        in_specs=[
            pl.BlockSpec((ROW_BLK, F), lambda i: (i, 0)),
            pl.BlockSpec((F, H), lambda i: (0, 0)),
        ],
        out_specs=pl.BlockSpec((ROW_BLK, H), lambda i: (i, 0)),
        out_shape=jax.ShapeDtypeStruct((N, H), jnp.float32),
    )(x, W_in)


def _make_spmm():
    mesh = plsc.VectorSubcoreMesh(core_axis_name="c", subcore_axis_name="s",
                                  num_cores=NC, num_subcores=NS)

    @functools.partial(
        pl.kernel,
        out_type=jax.ShapeDtypeStruct((NC, N_ACC, H), jnp.float32),
        mesh=mesh,
        scratch_types=[
            pltpu.VMEM((NCHUNK, CH), jnp.int32),
            pltpu.VMEM((NCHUNK, CH), jnp.int32),
            pltpu.VMEM((NBUF, CH, H), jnp.float32),
            pltpu.VMEM_SHARED((N_ACC, H), jnp.float32),
        ] + [pltpu.SemaphoreType.DMA] * NBUF,
        compiler_params=pltpu.CompilerParams(use_tc_tiling_on_sc=False),
    )
    def spmm(h_hbm, src_hbm, dst_hbm, zeros_hbm, out_hbm,
             src_v, dst_v, rows_v, acc, *sems):
        cid = lax.axis_index("c")
        sid = lax.axis_index("s")
        wid = sid * NC + cid
        pltpu.sync_copy(zeros_hbm, acc.at[pl.ds(sid * RPT, RPT)])
        pltpu.sync_copy(src_hbm.at[wid], src_v)
        pltpu.sync_copy(dst_hbm.at[wid], dst_v)
        plsc.subcore_barrier()

        for b in range(NBUF):
            pltpu.async_copy(h_hbm.at[src_v.at[b]], rows_v.at[b], sems[b])

        def body(i, carry):
            c0 = i * NBUF
            for b in range(NBUF):
                c = c0 + b
                pltpu.make_async_copy(
                    h_hbm.at[src_v.at[c]], rows_v.at[b], sems[b]).wait()
                pltpu.sync_copy(rows_v.at[b], acc.at[dst_v.at[c]], add=True)
                pltpu.async_copy(
                    h_hbm.at[src_v.at[c + NBUF]], rows_v.at[b], sems[b])
            return carry

        lax.fori_loop(0, NCHUNK // NBUF - 1, body, 0)

        for b in range(NBUF):
            c = NCHUNK - NBUF + b
            pltpu.make_async_copy(
                h_hbm.at[src_v.at[c]], rows_v.at[b], sems[b]).wait()
            pltpu.sync_copy(rows_v.at[b], acc.at[dst_v.at[c]], add=True)
        plsc.subcore_barrier()
        pltpu.sync_copy(acc.at[pl.ds(sid * RPT, RPT)],
                        out_hbm.at[cid, pl.ds(sid * RPT, RPT)])

    return spmm


_spmm_cache = []


def _spmm(*args):
    if not _spmm_cache:
        _spmm_cache.append(_make_spmm())
    return _spmm_cache[0](*args)


def _kan_body(p_ref, wc_ref, ws_ref, wo_ref, o_ref):
    agg = p_ref[0] + p_ref[1]
    y = jnp.zeros((ROW_BLK, H), jnp.float32)
    for g in range(G):
        kg = jnp.float32(g + 1)
        y = y + jnp.dot(jnp.cos(kg * agg), wc_ref[g],
                        preferred_element_type=jnp.float32)
        y = y + jnp.dot(jnp.sin(kg * agg), ws_ref[g],
                        preferred_element_type=jnp.float32)
    o = jnp.dot(y, wo_ref[...], preferred_element_type=jnp.float32)
    m = jnp.max(o, axis=-1, keepdims=True)
    lse = jnp.log(jnp.sum(jnp.exp(o - m), axis=-1, keepdims=True))
    o_ref[...] = (o - m) - lse


def _kan_out(partials, Wc, Ws, W_out):
    return pl.pallas_call(
        _kan_body,
        grid=(N // ROW_BLK,),
        in_specs=[
            pl.BlockSpec((NC, ROW_BLK, H), lambda i: (0, i, 0)),
            pl.BlockSpec((G, H, H), lambda i: (0, 0, 0)),
            pl.BlockSpec((G, H, H), lambda i: (0, 0, 0)),
            pl.BlockSpec((H, O), lambda i: (0, 0)),
        ],
        out_specs=pl.BlockSpec((ROW_BLK, O), lambda i: (i, 0)),
        out_shape=jax.ShapeDtypeStruct((N, O), jnp.float32),
    )(partials, Wc, Ws, W_out)


def kernel(x, adj, W_in, kan_coeffs, W_out):
    h = _lin_in(x, W_in)

    n_pad = E_PAD - E
    src = jnp.concatenate(
        [adj[0], jnp.zeros((n_pad,), jnp.int32)]).reshape(NW, NCHUNK, CH)
    dst = jnp.concatenate(
        [adj[1], jnp.full((n_pad,), N, jnp.int32)]).reshape(NW, NCHUNK, CH)
    zeros = jnp.zeros((RPT, H), jnp.float32)

    partials = _spmm(h, src, dst, zeros)

    Wc = jnp.transpose(kan_coeffs[0], (2, 1, 0))
    Ws = jnp.transpose(kan_coeffs[1], (2, 1, 0))
    return _kan_out(partials, Wc, Ws, W_out)
